# Initial kernel scaffold; baseline (speedup 1.0000x reference)
import functools

import jax
import jax.numpy as jnp
from jax import lax
from jax.experimental import pallas as pl
from jax.experimental.pallas import tpu as pltpu
from jax.experimental.pallas import tpu_sc as plsc

_E = 64
_K = 2
_CAP = 256
_SCALE = 1.0
_TBLK = 256
_NW = 32
_LANES = 16


def _router_body(ht_ref, gw_ref, d0_ref, d1_ref, w0_ref, w1_ref, carry_ref):
    E = gw_ref.shape[0]
    tblk = ht_ref.shape[1]

    @pl.when(pl.program_id(0) == 0)
    def _():
        carry_ref[...] = jnp.zeros_like(carry_ref)

    logits = jnp.dot(gw_ref[...], ht_ref[...],
                     preferred_element_type=jnp.float32)
    m = jnp.max(logits, axis=0, keepdims=True)
    ex = jnp.exp(logits - m)
    probs = ex / jnp.sum(ex, axis=0, keepdims=True)

    eio = lax.broadcasted_iota(jnp.int32, (E, tblk), 0)
    m1 = jnp.max(probs, axis=0, keepdims=True)
    idx1 = jnp.min(jnp.where(probs == m1, eio, E), axis=0, keepdims=True)
    sel1 = eio == idx1
    probs2 = jnp.where(sel1, -1.0, probs)
    m2 = jnp.max(probs2, axis=0, keepdims=True)
    idx2 = jnp.min(jnp.where(probs2 == m2, eio, E), axis=0, keepdims=True)
    sel2 = eio == idx2

    onehot = sel1.astype(jnp.float32) + sel2.astype(jnp.float32)
    io_r = lax.broadcasted_iota(jnp.int32, (tblk, tblk), 0)
    io_c = lax.broadcasted_iota(jnp.int32, (tblk, tblk), 1)
    upper = (io_r < io_c).astype(jnp.float32)
    pos = jnp.dot(onehot, upper,
                  preferred_element_type=jnp.float32) + carry_ref[...]

    slot1 = jnp.sum(jnp.where(sel1, pos, 0.0), axis=0, keepdims=True)
    slot2 = jnp.sum(jnp.where(sel2, pos, 0.0), axis=0, keepdims=True)
    s1 = (slot1 + 0.5).astype(jnp.int32)
    s2 = (slot2 + 0.5).astype(jnp.int32)
    dump = E * _CAP
    d0 = jnp.where(s1 < _CAP, idx1 * _CAP + s1, dump)
    d1 = jnp.where(s2 < _CAP, idx2 * _CAP + s2, dump)

    rows = d0_ref.shape[0]
    d0_ref[...] = jnp.broadcast_to(d0, (rows, tblk))
    d1_ref[...] = jnp.broadcast_to(d1, (rows, tblk))
    w0_ref[...] = jnp.broadcast_to(m1 * _SCALE, (rows, tblk))
    w1_ref[...] = jnp.broadcast_to(m2 * _SCALE, (rows, tblk))
    carry_ref[...] = carry_ref[...] + jnp.sum(onehot, axis=1, keepdims=True)


def _router(h_t, gate_w):
    H, T = h_t.shape
    E = gate_w.shape[0]
    nblk = T // _TBLK
    out_shape = [jax.ShapeDtypeStruct((8, T), jnp.int32),
                 jax.ShapeDtypeStruct((8, T), jnp.int32),
                 jax.ShapeDtypeStruct((8, T), jnp.float32),
                 jax.ShapeDtypeStruct((8, T), jnp.float32)]
    out_spec = pl.BlockSpec((8, _TBLK), lambda b: (0, b))
    return pl.pallas_call(
        _router_body,
        grid=(nblk,),
        in_specs=[pl.BlockSpec((H, _TBLK), lambda b: (0, b)),
                  pl.BlockSpec((E, H), lambda b: (0, 0))],
        out_specs=[out_spec, out_spec, out_spec, out_spec],
        out_shape=out_shape,
        scratch_shapes=[pltpu.VMEM((E, 1), jnp.float32)],
        compiler_params=pltpu.CompilerParams(
            dimension_semantics=("arbitrary",)),
    )(h_t, gate_w)


def _mlp_body(x_ref, gup_ref, dwn_ref, y_ref):
    I = dwn_ref.shape[1]
    gu = jnp.dot(x_ref[...], gup_ref[0],
                 preferred_element_type=jnp.float32)
    gate = gu[:, :I]
    up = gu[:, I:]
    inter = gate * lax.logistic(gate) * up
    y_ref[...] = jnp.dot(inter, dwn_ref[0],
                         preferred_element_type=jnp.float32)


def _expert_mlp(xbuf, gate_up_proj, down_proj):
    E, H, I2 = gate_up_proj.shape
    I = I2 // 2
    rows = xbuf.shape[0]
    return pl.pallas_call(
        _mlp_body,
        grid=(E,),
        in_specs=[pl.BlockSpec((_CAP, H), lambda e: (e, 0)),
                  pl.BlockSpec((1, H, I2), lambda e: (e, 0, 0)),
                  pl.BlockSpec((1, I, H), lambda e: (e, 0, 0))],
        out_specs=pl.BlockSpec((_CAP, H), lambda e: (e, 0)),
        out_shape=jax.ShapeDtypeStruct((rows, H), jnp.float32),
        compiler_params=pltpu.CompilerParams(
            dimension_semantics=("arbitrary",)),
    )(xbuf, gate_up_proj, down_proj)


def _make_dispatch(T, H, rows):
    tpw = T // _NW
    mesh = plsc.VectorSubcoreMesh(core_axis_name="c", subcore_axis_name="s")

    @functools.partial(
        pl.kernel, mesh=mesh,
        out_type=jax.ShapeDtypeStruct((rows, H), jnp.float32),
        scratch_types=[pltpu.VMEM((tpw, H), jnp.float32),
                       pltpu.VMEM((tpw,), jnp.int32),
                       pltpu.VMEM((tpw,), jnp.int32),
                       pltpu.SemaphoreType.DMA,
                       pltpu.SemaphoreType.DMA],
    )
    def dispatch(h_hbm, d0_hbm, d1_hbm, xbuf_hbm, hloc, d0v, d1v, s0, s1):
        wid = lax.axis_index("s") * 2 + lax.axis_index("c")
        base = wid * tpw
        pltpu.sync_copy(h_hbm.at[pl.ds(base, tpw)], hloc)
        pltpu.sync_copy(d0_hbm.at[0, pl.ds(base, tpw)], d0v)
        pltpu.sync_copy(d1_hbm.at[0, pl.ds(base, tpw)], d1v)
        c0 = pltpu.async_copy(hloc, xbuf_hbm.at[d0v], s0)
        c1 = pltpu.async_copy(hloc, xbuf_hbm.at[d1v], s1)
        c0.wait()
        c1.wait()

    return dispatch


def _make_combine(T, H, dump):
    tpw = T // _NW
    chunk = 32
    mesh = plsc.VectorSubcoreMesh(core_axis_name="c", subcore_axis_name="s")

    @functools.partial(
        pl.kernel, mesh=mesh,
        out_type=jax.ShapeDtypeStruct((T, H), jnp.float32),
        scratch_types=[pltpu.VMEM((chunk, H), jnp.float32),
                       pltpu.VMEM((chunk, H), jnp.float32),
                       pltpu.VMEM((chunk, H), jnp.float32),
                       pltpu.VMEM((chunk,), jnp.int32),
                       pltpu.VMEM((chunk,), jnp.int32),
                       pltpu.VMEM((chunk,), jnp.float32),
                       pltpu.VMEM((chunk,), jnp.float32),
                       pltpu.SemaphoreType.DMA,
                       pltpu.SemaphoreType.DMA],
    )
    def combine(ybuf_hbm, d0_hbm, d1_hbm, w0_hbm, w1_hbm, out_hbm,
                y0loc, y1loc, oloc, d0v, d1v, w0v, w1v, s0, s1):
        wid = lax.axis_index("s") * 2 + lax.axis_index("c")
        nsl = H // _LANES

        for c in range(tpw // chunk):
            base = wid * tpw + c * chunk
            pltpu.sync_copy(d0_hbm.at[0, pl.ds(base, chunk)], d0v)
            pltpu.sync_copy(d1_hbm.at[0, pl.ds(base, chunk)], d1v)
            pltpu.sync_copy(w0_hbm.at[0, pl.ds(base, chunk)], w0v)
            pltpu.sync_copy(w1_hbm.at[0, pl.ds(base, chunk)], w1v)
            g0 = pltpu.async_copy(ybuf_hbm.at[d0v], y0loc, s0)
            g1 = pltpu.async_copy(ybuf_hbm.at[d1v], y1loc, s1)
            g0.wait()
            g1.wait()

            def tok_body(i, carry):
                isplat = jnp.full((_LANES,), i, jnp.int32)
                d0s = plsc.load_gather(d0v, [isplat])
                d1s = plsc.load_gather(d1v, [isplat])
                w0s = plsc.load_gather(w0v, [isplat])
                w1s = plsc.load_gather(w1v, [isplat])
                v0 = d0s < dump
                v1 = d1s < dump
                zero = jnp.zeros((_LANES,), jnp.float32)
                for j in range(nsl):
                    sl = pl.ds(j * _LANES, _LANES)
                    a = y0loc[i, sl]
                    b = y1loc[i, sl]
                    oloc[i, sl] = (jnp.where(v0, a * w0s, zero)
                                   + jnp.where(v1, b * w1s, zero))
                return carry

            lax.fori_loop(0, chunk, tok_body, 0)
            pltpu.sync_copy(oloc, out_hbm.at[pl.ds(base, chunk)])

    return combine


def kernel(hidden_states, gate_w, gate_up_proj, down_proj):
    B, S, H = hidden_states.shape
    T = B * S
    E = gate_w.shape[0]
    rows = (E + 1) * _CAP

    h = hidden_states.reshape(T, H)
    h_t = h.T

    d0, d1, w0, w1 = _router(h_t, gate_w)
    xbuf = _make_dispatch(T, H, rows)(h, d0, d1)
    ybuf = _expert_mlp(xbuf, gate_up_proj, down_proj)
    out = _make_combine(T, H, E * _CAP)(ybuf, d0, d1, w0, w1)
    return out.reshape(B, S, H)


# speedup vs baseline: 7.0637x; 7.0637x over previous
import functools

import jax
import jax.numpy as jnp
from jax import lax
from jax.experimental import pallas as pl
from jax.experimental.pallas import tpu as pltpu
from jax.experimental.pallas import tpu_sc as plsc

_E = 64
_K = 2
_CAP = 256
_SCALE = 1.0
_TBLK = 256
_NW = 32
_LANES = 16


def _router_body(ht_ref, gw_ref, d0_ref, d1_ref, w0_ref, w1_ref, carry_ref):
    E = gw_ref.shape[0]
    tblk = ht_ref.shape[1]

    @pl.when(pl.program_id(0) == 0)
    def _():
        carry_ref[...] = jnp.zeros_like(carry_ref)

    logits = jnp.dot(gw_ref[...], ht_ref[...],
                     preferred_element_type=jnp.float32)
    m = jnp.max(logits, axis=0, keepdims=True)
    ex = jnp.exp(logits - m)
    probs = ex / jnp.sum(ex, axis=0, keepdims=True)

    eio = lax.broadcasted_iota(jnp.int32, (E, tblk), 0)
    m1 = jnp.max(probs, axis=0, keepdims=True)
    idx1 = jnp.min(jnp.where(probs == m1, eio, E), axis=0, keepdims=True)
    sel1 = eio == idx1
    probs2 = jnp.where(sel1, -1.0, probs)
    m2 = jnp.max(probs2, axis=0, keepdims=True)
    idx2 = jnp.min(jnp.where(probs2 == m2, eio, E), axis=0, keepdims=True)
    sel2 = eio == idx2

    onehot = sel1.astype(jnp.float32) + sel2.astype(jnp.float32)
    io_r = lax.broadcasted_iota(jnp.int32, (tblk, tblk), 0)
    io_c = lax.broadcasted_iota(jnp.int32, (tblk, tblk), 1)
    upper = (io_r < io_c).astype(jnp.float32)
    pos = jnp.dot(onehot, upper,
                  preferred_element_type=jnp.float32) + carry_ref[...]

    slot1 = jnp.sum(jnp.where(sel1, pos, 0.0), axis=0, keepdims=True)
    slot2 = jnp.sum(jnp.where(sel2, pos, 0.0), axis=0, keepdims=True)
    s1 = (slot1 + 0.5).astype(jnp.int32)
    s2 = (slot2 + 0.5).astype(jnp.int32)
    dump = E * _CAP
    ok1 = s1 < _CAP
    ok2 = s2 < _CAP
    d0 = jnp.where(ok1, idx1 * _CAP + s1, dump)
    d1 = jnp.where(ok2, idx2 * _CAP + s2, dump)
    w0 = jnp.where(ok1, m1 * _SCALE, 0.0)
    w1 = jnp.where(ok2, m2 * _SCALE, 0.0)

    rows = d0_ref.shape[0]
    d0_ref[...] = jnp.broadcast_to(d0, (rows, tblk))
    d1_ref[...] = jnp.broadcast_to(d1, (rows, tblk))
    w0_ref[...] = jnp.broadcast_to(w0, (rows, tblk))
    w1_ref[...] = jnp.broadcast_to(w1, (rows, tblk))
    carry_ref[...] = carry_ref[...] + jnp.sum(onehot, axis=1, keepdims=True)


def _router(h_t, gate_w):
    H, T = h_t.shape
    E = gate_w.shape[0]
    nblk = T // _TBLK
    out_shape = [jax.ShapeDtypeStruct((8, T), jnp.int32),
                 jax.ShapeDtypeStruct((8, T), jnp.int32),
                 jax.ShapeDtypeStruct((8, T), jnp.float32),
                 jax.ShapeDtypeStruct((8, T), jnp.float32)]
    out_spec = pl.BlockSpec((8, _TBLK), lambda b: (0, b))
    return pl.pallas_call(
        _router_body,
        grid=(nblk,),
        in_specs=[pl.BlockSpec((H, _TBLK), lambda b: (0, b)),
                  pl.BlockSpec((E, H), lambda b: (0, 0))],
        out_specs=[out_spec, out_spec, out_spec, out_spec],
        out_shape=out_shape,
        scratch_shapes=[pltpu.VMEM((E, 1), jnp.float32)],
        compiler_params=pltpu.CompilerParams(
            dimension_semantics=("arbitrary",)),
    )(h_t, gate_w)


def _mlp_body(num_experts, x_ref, gup_ref, dwn_ref, y_ref):
    I = dwn_ref.shape[1]
    gu = jnp.dot(x_ref[...], gup_ref[0],
                 preferred_element_type=jnp.float32)
    gate = gu[:, :I]
    up = gu[:, I:]
    inter = gate * lax.logistic(gate) * up
    y = jnp.dot(inter, dwn_ref[0],
                preferred_element_type=jnp.float32)
    y_ref[...] = jnp.where(pl.program_id(0) < num_experts, y, 0.0)


def _expert_mlp(xbuf, gate_up_proj, down_proj):
    E, H, I2 = gate_up_proj.shape
    I = I2 // 2
    rows = xbuf.shape[0]
    return pl.pallas_call(
        functools.partial(_mlp_body, E),
        grid=(E + 1,),
        in_specs=[pl.BlockSpec((_CAP, H), lambda e: (e, 0)),
                  pl.BlockSpec((1, H, I2),
                               lambda e: (jnp.minimum(e, E - 1), 0, 0)),
                  pl.BlockSpec((1, I, H),
                               lambda e: (jnp.minimum(e, E - 1), 0, 0))],
        out_specs=pl.BlockSpec((_CAP, H), lambda e: (e, 0)),
        out_shape=jax.ShapeDtypeStruct((rows, H), jnp.float32),
        compiler_params=pltpu.CompilerParams(
            dimension_semantics=("arbitrary",)),
    )(xbuf, gate_up_proj, down_proj)


def _make_dispatch(T, H, rows):
    tpw = T // _NW
    mesh = plsc.VectorSubcoreMesh(core_axis_name="c", subcore_axis_name="s")

    @functools.partial(
        pl.kernel, mesh=mesh,
        out_type=jax.ShapeDtypeStruct((rows, H), jnp.float32),
        scratch_types=[pltpu.VMEM((tpw, H), jnp.float32),
                       pltpu.VMEM((tpw,), jnp.int32),
                       pltpu.VMEM((tpw,), jnp.int32),
                       pltpu.SemaphoreType.DMA,
                       pltpu.SemaphoreType.DMA],
    )
    def dispatch(h_hbm, d0_hbm, d1_hbm, xbuf_hbm, hloc, d0v, d1v, s0, s1):
        wid = lax.axis_index("s") * 2 + lax.axis_index("c")
        base = wid * tpw
        pltpu.sync_copy(h_hbm.at[pl.ds(base, tpw)], hloc)
        pltpu.sync_copy(d0_hbm.at[0, pl.ds(base, tpw)], d0v)
        pltpu.sync_copy(d1_hbm.at[0, pl.ds(base, tpw)], d1v)
        c0 = pltpu.async_copy(hloc, xbuf_hbm.at[d0v], s0)
        c1 = pltpu.async_copy(hloc, xbuf_hbm.at[d1v], s1)
        c0.wait()
        c1.wait()

    return dispatch


def _make_combine(T, H, dump):
    tpw = T // _NW
    chunk = 32
    mesh = plsc.VectorSubcoreMesh(core_axis_name="c", subcore_axis_name="s")

    @functools.partial(
        pl.kernel, mesh=mesh,
        out_type=jax.ShapeDtypeStruct((T, H), jnp.float32),
        scratch_types=[pltpu.VMEM((chunk, H), jnp.float32),
                       pltpu.VMEM((chunk, H), jnp.float32),
                       pltpu.VMEM((chunk, H), jnp.float32),
                       pltpu.VMEM((chunk,), jnp.int32),
                       pltpu.VMEM((chunk,), jnp.int32),
                       pltpu.VMEM((chunk,), jnp.float32),
                       pltpu.VMEM((chunk,), jnp.float32),
                       pltpu.SemaphoreType.DMA,
                       pltpu.SemaphoreType.DMA],
    )
    def combine(ybuf_hbm, d0_hbm, d1_hbm, w0_hbm, w1_hbm, out_hbm,
                y0loc, y1loc, oloc, d0v, d1v, w0v, w1v, s0, s1):
        wid = lax.axis_index("s") * 2 + lax.axis_index("c")
        nsl = H // _LANES

        for c in range(tpw // chunk):
            base = wid * tpw + c * chunk
            pltpu.sync_copy(d0_hbm.at[0, pl.ds(base, chunk)], d0v)
            pltpu.sync_copy(d1_hbm.at[0, pl.ds(base, chunk)], d1v)
            pltpu.sync_copy(w0_hbm.at[0, pl.ds(base, chunk)], w0v)
            pltpu.sync_copy(w1_hbm.at[0, pl.ds(base, chunk)], w1v)
            g0 = pltpu.async_copy(ybuf_hbm.at[d0v], y0loc, s0)
            g1 = pltpu.async_copy(ybuf_hbm.at[d1v], y1loc, s1)
            g0.wait()
            g1.wait()

            for g in range(chunk // _LANES):
                w0g = w0v[pl.ds(g * _LANES, _LANES)]
                w1g = w1v[pl.ds(g * _LANES, _LANES)]
                for i2 in range(_LANES):
                    tok = g * _LANES + i2
                    w0s = jnp.full((_LANES,), w0g[i2], jnp.float32)
                    w1s = jnp.full((_LANES,), w1g[i2], jnp.float32)

                    def j_body(j, carry, tok=tok, w0s=w0s, w1s=w1s):
                        sl = pl.ds(j * _LANES, _LANES)
                        oloc[tok, sl] = (y0loc[tok, sl] * w0s
                                         + y1loc[tok, sl] * w1s)
                        return carry

                    lax.fori_loop(0, nsl, j_body, 0)
            pltpu.sync_copy(oloc, out_hbm.at[pl.ds(base, chunk)])

    return combine


def kernel(hidden_states, gate_w, gate_up_proj, down_proj):
    B, S, H = hidden_states.shape
    T = B * S
    E = gate_w.shape[0]
    rows = (E + 1) * _CAP

    h = hidden_states.reshape(T, H)
    h_t = h.T

    d0, d1, w0, w1 = _router(h_t, gate_w)
    xbuf = _make_dispatch(T, H, rows)(h, d0, d1)
    ybuf = _expert_mlp(xbuf, gate_up_proj, down_proj)
    out = _make_combine(T, H, E * _CAP)(ybuf, d0, d1, w0, w1)
    return out.reshape(B, S, H)


# speedup vs baseline: 7.0759x; 1.0017x over previous
import functools

import jax
import jax.numpy as jnp
from jax import lax
from jax.experimental import pallas as pl
from jax.experimental.pallas import tpu as pltpu
from jax.experimental.pallas import tpu_sc as plsc

_E = 64
_K = 2
_CAP = 256
_SCALE = 1.0
_TBLK = 256
_NW = 32
_LANES = 16


def _router_body(ht_ref, gw_ref, d0_ref, d1_ref, w0_ref, w1_ref, carry_ref):
    E = gw_ref.shape[0]
    tblk = ht_ref.shape[1]

    @pl.when(pl.program_id(0) == 0)
    def _():
        carry_ref[...] = jnp.zeros_like(carry_ref)

    logits = jnp.dot(gw_ref[...], ht_ref[...],
                     preferred_element_type=jnp.float32)
    m = jnp.max(logits, axis=0, keepdims=True)
    ex = jnp.exp(logits - m)
    probs = ex / jnp.sum(ex, axis=0, keepdims=True)

    eio = lax.broadcasted_iota(jnp.int32, (E, tblk), 0)
    m1 = jnp.max(probs, axis=0, keepdims=True)
    idx1 = jnp.min(jnp.where(probs == m1, eio, E), axis=0, keepdims=True)
    sel1 = eio == idx1
    probs2 = jnp.where(sel1, -1.0, probs)
    m2 = jnp.max(probs2, axis=0, keepdims=True)
    idx2 = jnp.min(jnp.where(probs2 == m2, eio, E), axis=0, keepdims=True)
    sel2 = eio == idx2

    onehot = sel1.astype(jnp.float32) + sel2.astype(jnp.float32)
    io_r = lax.broadcasted_iota(jnp.int32, (tblk, tblk), 0)
    io_c = lax.broadcasted_iota(jnp.int32, (tblk, tblk), 1)
    upper = (io_r < io_c).astype(jnp.float32)
    pos = jnp.dot(onehot, upper,
                  preferred_element_type=jnp.float32) + carry_ref[...]

    slot1 = jnp.sum(jnp.where(sel1, pos, 0.0), axis=0, keepdims=True)
    slot2 = jnp.sum(jnp.where(sel2, pos, 0.0), axis=0, keepdims=True)
    s1 = (slot1 + 0.5).astype(jnp.int32)
    s2 = (slot2 + 0.5).astype(jnp.int32)
    dump = E * _CAP
    ok1 = s1 < _CAP
    ok2 = s2 < _CAP
    d0 = jnp.where(ok1, idx1 * _CAP + s1, dump)
    d1 = jnp.where(ok2, idx2 * _CAP + s2, dump)
    w0 = jnp.where(ok1, m1 * _SCALE, 0.0)
    w1 = jnp.where(ok2, m2 * _SCALE, 0.0)

    rows = d0_ref.shape[0]
    d0_ref[...] = jnp.broadcast_to(d0, (rows, tblk))
    d1_ref[...] = jnp.broadcast_to(d1, (rows, tblk))
    w0_ref[...] = jnp.broadcast_to(w0, (rows, tblk))
    w1_ref[...] = jnp.broadcast_to(w1, (rows, tblk))
    carry_ref[...] = carry_ref[...] + jnp.sum(onehot, axis=1, keepdims=True)


def _router(h_t, gate_w):
    H, T = h_t.shape
    E = gate_w.shape[0]
    nblk = T // _TBLK
    out_shape = [jax.ShapeDtypeStruct((8, T), jnp.int32),
                 jax.ShapeDtypeStruct((8, T), jnp.int32),
                 jax.ShapeDtypeStruct((8, T), jnp.float32),
                 jax.ShapeDtypeStruct((8, T), jnp.float32)]
    out_spec = pl.BlockSpec((8, _TBLK), lambda b: (0, b))
    return pl.pallas_call(
        _router_body,
        grid=(nblk,),
        in_specs=[pl.BlockSpec((H, _TBLK), lambda b: (0, b)),
                  pl.BlockSpec((E, H), lambda b: (0, 0))],
        out_specs=[out_spec, out_spec, out_spec, out_spec],
        out_shape=out_shape,
        scratch_shapes=[pltpu.VMEM((E, 1), jnp.float32)],
        compiler_params=pltpu.CompilerParams(
            dimension_semantics=("arbitrary",)),
    )(h_t, gate_w)


def _mlp_body(num_experts, x_ref, gup_ref, dwn_ref, y_ref):
    I = dwn_ref.shape[1]
    xb = x_ref[...].astype(jnp.bfloat16)
    gu = jnp.dot(xb, gup_ref[0].astype(jnp.bfloat16),
                 preferred_element_type=jnp.float32)
    gate = gu[:, :I]
    up = gu[:, I:]
    inter = (gate * lax.logistic(gate) * up).astype(jnp.bfloat16)
    y = jnp.dot(inter, dwn_ref[0].astype(jnp.bfloat16),
                preferred_element_type=jnp.float32)
    y_ref[...] = jnp.where(pl.program_id(0) < num_experts, y, 0.0)


def _expert_mlp(xbuf, gate_up_proj, down_proj):
    E, H, I2 = gate_up_proj.shape
    I = I2 // 2
    rows = xbuf.shape[0]
    return pl.pallas_call(
        functools.partial(_mlp_body, E),
        grid=(E + 1,),
        in_specs=[pl.BlockSpec((_CAP, H), lambda e: (e, 0)),
                  pl.BlockSpec((1, H, I2),
                               lambda e: (jnp.minimum(e, E - 1), 0, 0)),
                  pl.BlockSpec((1, I, H),
                               lambda e: (jnp.minimum(e, E - 1), 0, 0))],
        out_specs=pl.BlockSpec((_CAP, H), lambda e: (e, 0)),
        out_shape=jax.ShapeDtypeStruct((rows, H), jnp.float32),
        compiler_params=pltpu.CompilerParams(
            dimension_semantics=("arbitrary",)),
    )(xbuf, gate_up_proj, down_proj)


def _make_dispatch(T, H, rows):
    tpw = T // _NW
    mesh = plsc.VectorSubcoreMesh(core_axis_name="c", subcore_axis_name="s")

    @functools.partial(
        pl.kernel, mesh=mesh,
        out_type=jax.ShapeDtypeStruct((rows, H), jnp.float32),
        scratch_types=[pltpu.VMEM((tpw, H), jnp.float32),
                       pltpu.VMEM((tpw,), jnp.int32),
                       pltpu.VMEM((tpw,), jnp.int32),
                       pltpu.SemaphoreType.DMA,
                       pltpu.SemaphoreType.DMA],
    )
    def dispatch(h_hbm, d0_hbm, d1_hbm, xbuf_hbm, hloc, d0v, d1v, s0, s1):
        wid = lax.axis_index("s") * 2 + lax.axis_index("c")
        base = wid * tpw
        pltpu.sync_copy(h_hbm.at[pl.ds(base, tpw)], hloc)
        pltpu.sync_copy(d0_hbm.at[0, pl.ds(base, tpw)], d0v)
        pltpu.sync_copy(d1_hbm.at[0, pl.ds(base, tpw)], d1v)
        c0 = pltpu.async_copy(hloc, xbuf_hbm.at[d0v], s0)
        c1 = pltpu.async_copy(hloc, xbuf_hbm.at[d1v], s1)
        c0.wait()
        c1.wait()

    return dispatch


def _make_combine(T, H, dump):
    tpw = T // _NW
    chunk = 32
    mesh = plsc.VectorSubcoreMesh(core_axis_name="c", subcore_axis_name="s")

    @functools.partial(
        pl.kernel, mesh=mesh,
        out_type=jax.ShapeDtypeStruct((T, H), jnp.float32),
        scratch_types=[pltpu.VMEM((chunk, H), jnp.float32),
                       pltpu.VMEM((chunk, H), jnp.float32),
                       pltpu.VMEM((chunk, H), jnp.float32),
                       pltpu.VMEM((chunk,), jnp.int32),
                       pltpu.VMEM((chunk,), jnp.int32),
                       pltpu.VMEM((chunk,), jnp.float32),
                       pltpu.VMEM((chunk,), jnp.float32),
                       pltpu.SemaphoreType.DMA,
                       pltpu.SemaphoreType.DMA],
    )
    def combine(ybuf_hbm, d0_hbm, d1_hbm, w0_hbm, w1_hbm, out_hbm,
                y0loc, y1loc, oloc, d0v, d1v, w0v, w1v, s0, s1):
        wid = lax.axis_index("s") * 2 + lax.axis_index("c")
        nsl = H // _LANES

        for c in range(tpw // chunk):
            base = wid * tpw + c * chunk
            pltpu.sync_copy(d0_hbm.at[0, pl.ds(base, chunk)], d0v)
            pltpu.sync_copy(d1_hbm.at[0, pl.ds(base, chunk)], d1v)
            pltpu.sync_copy(w0_hbm.at[0, pl.ds(base, chunk)], w0v)
            pltpu.sync_copy(w1_hbm.at[0, pl.ds(base, chunk)], w1v)
            g0 = pltpu.async_copy(ybuf_hbm.at[d0v], y0loc, s0)
            g1 = pltpu.async_copy(ybuf_hbm.at[d1v], y1loc, s1)
            g0.wait()
            g1.wait()

            for g in range(chunk // _LANES):
                w0g = w0v[pl.ds(g * _LANES, _LANES)]
                w1g = w1v[pl.ds(g * _LANES, _LANES)]
                for i2 in range(_LANES):
                    tok = g * _LANES + i2
                    w0s = jnp.full((_LANES,), w0g[i2], jnp.float32)
                    w1s = jnp.full((_LANES,), w1g[i2], jnp.float32)

                    def j_body(j, carry, tok=tok, w0s=w0s, w1s=w1s):
                        sl = pl.ds(j * _LANES, _LANES)
                        oloc[tok, sl] = (y0loc[tok, sl] * w0s
                                         + y1loc[tok, sl] * w1s)
                        return carry

                    lax.fori_loop(0, nsl, j_body, 0)
            pltpu.sync_copy(oloc, out_hbm.at[pl.ds(base, chunk)])

    return combine


def kernel(hidden_states, gate_w, gate_up_proj, down_proj):
    B, S, H = hidden_states.shape
    T = B * S
    E = gate_w.shape[0]
    rows = (E + 1) * _CAP

    h = hidden_states.reshape(T, H)
    h_t = h.T

    d0, d1, w0, w1 = _router(h_t, gate_w)
    xbuf = _make_dispatch(T, H, rows)(h, d0, d1)
    ybuf = _expert_mlp(xbuf, gate_up_proj, down_proj)
    out = _make_combine(T, H, E * _CAP)(ybuf, d0, d1, w0, w1)
    return out.reshape(B, S, H)
